# baseline (speedup 1.0000x reference)
import functools

import jax
import jax.numpy as jnp
from jax import lax
from jax.experimental import pallas as pl
from jax.experimental.pallas import tpu as pltpu

H = 128
_NNODES = {"drug": 10000, "protein": 10000, "disease": 2048}
_RELS = [
    ("e_d_t_dr", "disease", "drug"),
    ("e_d_m_dr", "disease", "drug"),
    ("e_d_p", "disease", "protein"),
    ("e_dr_t_d", "drug", "disease"),
    ("e_dr_m_d", "drug", "disease"),
    ("e_p_d", "protein", "disease"),
    ("e_DDI", "drug", "drug"),
    ("e_PPI", "protein", "protein"),
]


def _bn_relu(z, g, b):
    m = jnp.mean(z, 0)
    v = jnp.mean((z - m) ** 2, 0)
    return jax.nn.relu((z - m) * lax.rsqrt(v + 1e-5) * g + b)


def _head_body(feat_ref, W1, b1, g1, be1, W2, b2, g2, be2, W3, b3, g3, be3,
               Wout, bout, out_ref):
    z = jnp.dot(feat_ref[...], W1[...], preferred_element_type=jnp.float32) + b1[...]
    o = _bn_relu(z, g1[...], be1[...])
    z = jnp.dot(o, W2[...], preferred_element_type=jnp.float32) + b2[...]
    o = _bn_relu(z, g2[...], be2[...])
    z = jnp.dot(o, W3[...], preferred_element_type=jnp.float32) + b3[...]
    o = _bn_relu(z, g3[...], be3[...])
    z = jnp.dot(o, Wout[...], preferred_element_type=jnp.float32) + bout[...]
    out_ref[...] = jax.nn.sigmoid(z)


def _head(feat, W1, b1, g1, be1, W2, b2, g2, be2, W3, b3, g3, be3, Wout, bout):
    B = feat.shape[0]
    return pl.pallas_call(
        _head_body,
        out_shape=jax.ShapeDtypeStruct((B, 1), jnp.float32),
    )(feat, W1, b1, g1, be1, W2, b2, g2, be2, W3, b3, g3, be3, Wout, bout)


def _sage(h_src, h_dst, edges, n_dst, Wp, bp, Ws, Wn, b):
    m = jax.nn.relu(h_src @ Wp + bp)
    msgs = m[edges[0]]
    neigh = jax.ops.segment_max(msgs, edges[1], num_segments=n_dst)
    neigh = jnp.where(jnp.isneginf(neigh), 0.0, neigh)
    return jax.nn.relu(h_dst @ Ws + neigh @ Wn + b)


def _hgcn(h, edges, Wp, bp, Ws, Wn, bs):
    out = {nt: jnp.zeros((_NNODES[nt], H), jnp.float32) for nt in _NNODES}
    for i, (name, st, dt) in enumerate(_RELS):
        out[dt] = out[dt] + _sage(h[st], h[dt], edges[name], _NNODES[dt],
                                  Wp[i], bp[i], Ws[i], Wn[i], bs[i])
    return out


def kernel(x_dr, x_p, finger_feats, seq_feats, disease_feat, e_d_t_dr, e_d_m_dr,
           e_d_p, e_dr_t_d, e_dr_m_d, e_p_d, e_DDI, e_PPI, W_fing, b_fing,
           W_seq, b_seq, W_dis, b_dis, Wp, bp, Ws, Wn, bs, W1, b1, g1, be1,
           W2, b2, g2, be2, W3, b3, g3, be3, Wout, bout):
    edges = {"e_d_t_dr": e_d_t_dr, "e_d_m_dr": e_d_m_dr, "e_d_p": e_d_p,
             "e_dr_t_d": e_dr_t_d, "e_dr_m_d": e_dr_m_d, "e_p_d": e_p_d,
             "e_DDI": e_DDI, "e_PPI": e_PPI}
    h_dr_f = jax.nn.relu(finger_feats @ W_fing + b_fing)
    h_p_s = jax.nn.relu(seq_feats @ W_seq + b_seq)
    h_d = jax.nn.relu(disease_feat @ W_dis + b_dis)
    h0 = {"drug": h_dr_f, "protein": h_p_s, "disease": h_d}
    h1 = _hgcn(h0, edges, Wp, bp, Ws, Wn, bs)
    h2 = _hgcn(h1, edges, Wp, bp, Ws, Wn, bs)
    dr_new = jnp.concatenate([h_dr_f, h1["drug"], h2["drug"]], axis=1)
    p_new = jnp.concatenate([h_p_s, h1["protein"], h2["protein"]], axis=1)
    feat = jnp.concatenate([dr_new[x_dr], p_new[x_p]], axis=1)
    return _head(feat, W1, b1, g1, be1, W2, b2, g2, be2, W3, b3, g3, be3,
                 Wout, bout)
